# baseline (speedup 1.0000x reference)
import jax
import jax.numpy as jnp
from jax.experimental import pallas as pl
from jax.experimental.pallas import tpu as pltpu

_BATCH = 4096
_HIST = 200
_NCHUNK = 8
_CHUNK = _BATCH // _NCHUNK
_NBUF = 4


def _copy_body(x_hbm, o_hbm, buf, in_sems, out_sems):
    def in_copy(c):
        rows = pl.ds(c * _CHUNK, _CHUNK)
        return pltpu.make_async_copy(
            x_hbm.at[rows], buf.at[c % _NBUF], in_sems.at[c % _NBUF])

    def out_copy(c):
        rows = pl.ds(c * _CHUNK, _CHUNK)
        return pltpu.make_async_copy(
            buf.at[c % _NBUF], o_hbm.at[rows], out_sems.at[c % _NBUF])

    for c in range(min(_NBUF - 1, _NCHUNK)):
        in_copy(c).start()
    for c in range(_NCHUNK):
        if c + _NBUF - 1 < _NCHUNK:
            if c >= 1:
                out_copy(c - 1).wait()
            in_copy(c + _NBUF - 1).start()
        elif c >= 1:
            out_copy(c - 1).wait()
        in_copy(c).wait()
        out_copy(c).start()
    out_copy(_NCHUNK - 1).wait()


def kernel(z, x, W_h, b_h, emb):
    del z, W_h, b_h, emb
    return pl.pallas_call(
        _copy_body,
        out_shape=jax.ShapeDtypeStruct((_BATCH, _HIST), jnp.int32),
        in_specs=[pl.BlockSpec(memory_space=pl.MemorySpace.ANY)],
        out_specs=pl.BlockSpec(memory_space=pl.MemorySpace.ANY),
        scratch_shapes=[
            pltpu.VMEM((_NBUF, _CHUNK, _HIST), jnp.int32),
            pltpu.SemaphoreType.DMA((_NBUF,)),
            pltpu.SemaphoreType.DMA((_NBUF,)),
        ],
    )(x)
